# baseline (speedup 1.0000x reference)
import functools

import jax
import jax.numpy as jnp
from jax import lax
from jax.experimental import pallas as pl
from jax.experimental.pallas import tpu as pltpu
from jax.experimental.pallas import tpu_sc as plsc

NC, NS, L = 2, 16, 16
NW = NC * NS
CB = 128


def _sc_mesh():
    return plsc.VectorSubcoreMesh(
        core_axis_name="c", subcore_axis_name="s",
        num_cores=NC, num_subcores=NS)


def _build_sc_prep(n_pad, n_chunks, dummy):
    cw = n_chunks // NW

    @functools.partial(
        pl.kernel,
        out_type=[
            jax.ShapeDtypeStruct((n_chunks, CB), jnp.int32),
            jax.ShapeDtypeStruct((n_chunks, CB), jnp.int32),
            jax.ShapeDtypeStruct((NC, n_pad), jnp.float32),
        ],
        mesh=_sc_mesh(),
        scratch_types=[
            pltpu.VMEM((cw, CB), jnp.int32),
            pltpu.VMEM((cw, CB), jnp.int32),
            pltpu.VMEM((cw, CB), jnp.int32),
            pltpu.VMEM((cw, CB), jnp.int32),
            pltpu.VMEM((CB,), jnp.float32),
            pltpu.VMEM((2048,), jnp.float32),
            pltpu.VMEM_SHARED((n_pad,), jnp.float32),
        ],
    )
    def prep(rows_hbm, cols_hbm, rowm_hbm, colm_hbm, degp_hbm,
             rbuf, cbuf, rm, cm, ones_b, zbuf, deg_sh):
        cidx = lax.axis_index("c")
        sidx = lax.axis_index("s")
        wid = sidx * NC + cidx
        base = wid * cw

        pltpu.sync_copy(rows_hbm.at[pl.ds(base, cw)], rbuf)
        pltpu.sync_copy(cols_hbm.at[pl.ds(base, cw)], cbuf)

        for k in range(CB // L):
            ones_b[pl.ds(k * L, L)] = jnp.ones((L,), jnp.float32)

        @pl.when(sidx == 0)
        def _zero_deg():
            def zb(i, carry):
                zbuf[pl.ds(i * L, L)] = jnp.zeros((L,), jnp.float32)
                return carry
            lax.fori_loop(0, 2048 // L, zb, 0)

            def zc(t, carry):
                pltpu.sync_copy(zbuf, deg_sh.at[pl.ds(t * 2048, 2048)])
                return carry
            lax.fori_loop(0, n_pad // 2048, zc, 0)

        def remap(j, carry):
            for k in range(CB // L):
                dvec = dummy + k * L + lax.broadcasted_iota(jnp.int32, (L,), 0)
                r = rbuf[j, pl.ds(k * L, L)]
                c2 = cbuf[j, pl.ds(k * L, L)]
                m = r == c2
                rm[j, pl.ds(k * L, L)] = jnp.where(m, dvec, r)
                cm[j, pl.ds(k * L, L)] = jnp.where(m, dvec, c2)
            return carry
        lax.fori_loop(0, cw, remap, 0)

        pltpu.sync_copy(rm, rowm_hbm.at[pl.ds(base, cw)])
        pltpu.sync_copy(cm, colm_hbm.at[pl.ds(base, cw)])

        plsc.subcore_barrier()

        def scat(j, carry):
            pltpu.sync_copy(ones_b, deg_sh.at[rm.at[j]], add=True)
            return carry
        lax.fori_loop(0, cw, scat, 0)

        plsc.subcore_barrier()

        @pl.when(sidx == 0)
        def _drain():
            pltpu.sync_copy(deg_sh, degp_hbm.at[cidx])

    return prep


def _build_sc_prop(n_pad, n_chunks):
    cw = n_chunks // NW
    ch = cw // 2
    rpt = n_pad // NS

    @functools.partial(
        pl.kernel,
        out_type=jax.ShapeDtypeStruct((NC, n_pad, 128), jnp.float32),
        mesh=_sc_mesh(),
        scratch_types=[
            pltpu.VMEM((ch, CB), jnp.int32),
            pltpu.VMEM((ch, CB), jnp.int32),
            pltpu.VMEM((2, CB, 128), jnp.float32),
            pltpu.VMEM_SHARED((n_pad, 128), jnp.float32),
            pltpu.SemaphoreType.DMA,
            pltpu.SemaphoreType.DMA,
            pltpu.SemaphoreType.DMA,
            pltpu.SemaphoreType.DMA,
        ],
    )
    def prop(g_hbm, rowm_hbm, colm_hbm, part_hbm, rbuf, cbuf, msg,
             acc_sh, sem0, sem1, ssem0, ssem1):
        cidx = lax.axis_index("c")
        sidx = lax.axis_index("s")
        wid = sidx * NC + cidx
        base = wid * cw

        pltpu.sync_copy(rowm_hbm.at[pl.ds(base, ch)], rbuf)
        pltpu.sync_copy(colm_hbm.at[pl.ds(base, ch)], cbuf)
        pltpu.async_copy(g_hbm.at[rbuf.at[0]], msg.at[0], sem0)

        def zb(i, carry):
            for k in range(128 // L):
                msg[1, i, pl.ds(k * L, L)] = jnp.zeros((L,), jnp.float32)
            return carry
        lax.fori_loop(0, CB, zb, 0)

        def zc(t, carry):
            pltpu.sync_copy(msg.at[1], acc_sh.at[pl.ds(sidx * rpt + t * CB, CB)])
            return carry
        lax.fori_loop(0, rpt // CB, zc, 0)

        plsc.subcore_barrier()

        for h in range(2):
            if h:
                pltpu.make_async_copy(
                    msg.at[0], acc_sh.at[cbuf.at[ch - 2]], ssem0).wait()
                pltpu.make_async_copy(
                    msg.at[1], acc_sh.at[cbuf.at[ch - 1]], ssem1).wait()
                pltpu.sync_copy(rowm_hbm.at[pl.ds(base + h * ch, ch)], rbuf)
                pltpu.sync_copy(colm_hbm.at[pl.ds(base + h * ch, ch)], cbuf)
                pltpu.async_copy(g_hbm.at[rbuf.at[0]], msg.at[0], sem0)
            pltpu.async_copy(g_hbm.at[rbuf.at[1]], msg.at[1], sem1)

            def body(jj, carry):
                j = jj * 2
                pltpu.make_async_copy(
                    g_hbm.at[rbuf.at[j]], msg.at[0], sem0).wait()
                pltpu.make_async_copy(
                    msg.at[0], acc_sh.at[cbuf.at[j]], ssem0).start(add=True)
                pltpu.make_async_copy(
                    g_hbm.at[rbuf.at[j + 1]], msg.at[1], sem1).wait()
                pltpu.make_async_copy(
                    msg.at[1], acc_sh.at[cbuf.at[j + 1]], ssem1).start(add=True)

                @pl.when(jj < ch // 2 - 1)
                def _():
                    pltpu.make_async_copy(
                        msg.at[0], acc_sh.at[cbuf.at[j]], ssem0).wait()
                    pltpu.async_copy(
                        g_hbm.at[rbuf.at[j + 2]], msg.at[0], sem0)
                    pltpu.make_async_copy(
                        msg.at[1], acc_sh.at[cbuf.at[j + 1]], ssem1).wait()
                    pltpu.async_copy(
                        g_hbm.at[rbuf.at[j + 3]], msg.at[1], sem1)
                return carry
            lax.fori_loop(0, ch // 2, body, 0)

        pltpu.make_async_copy(
            msg.at[0], acc_sh.at[cbuf.at[ch - 2]], ssem0).wait()
        pltpu.make_async_copy(
            msg.at[1], acc_sh.at[cbuf.at[ch - 1]], ssem1).wait()

        plsc.subcore_barrier()

        pltpu.sync_copy(acc_sh.at[pl.ds(sidx * rpt, rpt)],
                        part_hbm.at[cidx, pl.ds(sidx * rpt, rpt)])

    return prop


def _mm_scale_call(degp3, x_pad, w, n, bm=1024):
    n_pad = x_pad.shape[0]
    d_out = w.shape[1]

    def body(d_ref, x_ref, w_ref, g_ref, dis_ref):
        i = pl.program_id(0)
        deg = d_ref[0] + d_ref[1] + 1.0
        idx = lax.broadcasted_iota(jnp.int32, (bm, 1), 0) + i * bm
        dis = jnp.where(idx < n, lax.rsqrt(deg), 0.0)
        dis_ref[...] = dis
        g_ref[...] = jnp.dot(x_ref[...], w_ref[...],
                             preferred_element_type=jnp.float32) * dis

    return pl.pallas_call(
        body,
        grid=(n_pad // bm,),
        in_specs=[
            pl.BlockSpec((NC, bm, 1), lambda i: (0, i, 0)),
            pl.BlockSpec((bm, x_pad.shape[1]), lambda i: (i, 0)),
            pl.BlockSpec(w.shape, lambda i: (0, 0)),
        ],
        out_specs=[
            pl.BlockSpec((bm, d_out), lambda i: (i, 0)),
            pl.BlockSpec((bm, 1), lambda i: (i, 0)),
        ],
        out_shape=[
            jax.ShapeDtypeStruct((n_pad, d_out), jnp.float32),
            jax.ShapeDtypeStruct((n_pad, 1), jnp.float32),
        ],
    )(degp3, x_pad, w)


def _mid_call(part, g1, dis_col, b1r, w2, bm=1024):
    n_pad = g1.shape[0]
    d_out = w2.shape[1]

    def body(p_ref, g_ref, d_ref, b_ref, w_ref, o_ref):
        agg = p_ref[0] + p_ref[1] + g_ref[...]
        h1 = jnp.maximum(agg * d_ref[...] + b_ref[...], 0.0)
        o_ref[...] = jnp.dot(h1, w_ref[...],
                             preferred_element_type=jnp.float32) * d_ref[...]

    return pl.pallas_call(
        body,
        grid=(n_pad // bm,),
        in_specs=[
            pl.BlockSpec((NC, bm, 128), lambda i: (0, i, 0)),
            pl.BlockSpec((bm, 128), lambda i: (i, 0)),
            pl.BlockSpec((bm, 1), lambda i: (i, 0)),
            pl.BlockSpec((1, 128), lambda i: (0, 0)),
            pl.BlockSpec(w2.shape, lambda i: (0, 0)),
        ],
        out_specs=pl.BlockSpec((bm, d_out), lambda i: (i, 0)),
        out_shape=jax.ShapeDtypeStruct((n_pad, d_out), jnp.float32),
    )(part, g1, dis_col, b1r, w2)


def _fin_call(part, g2, dis_col, b2r, n, bm=1024):
    n_pad = g2.shape[0]

    def body(p_ref, g_ref, d_ref, b_ref, o_ref):
        o_ref[...] = (p_ref[0] + p_ref[1] + g_ref[...]) * d_ref[...] + b_ref[...]

    return pl.pallas_call(
        body,
        grid=(pl.cdiv(n, bm),),
        in_specs=[
            pl.BlockSpec((NC, bm, 128), lambda i: (0, i, 0)),
            pl.BlockSpec((bm, 128), lambda i: (i, 0)),
            pl.BlockSpec((bm, 1), lambda i: (i, 0)),
            pl.BlockSpec((1, 128), lambda i: (0, 0)),
        ],
        out_specs=pl.BlockSpec((bm, 128), lambda i: (i, 0)),
        out_shape=jax.ShapeDtypeStruct((n, 128), jnp.float32),
    )(part, g2, dis_col, b2r)


def kernel(x, edge_index, cache_name, W1, b1, W2, b2):
    del cache_name
    n, din = x.shape
    e = edge_index.shape[1]
    dummy = n
    n_pad = ((n + CB + 1023) // 1024) * 1024

    cw = -(-(-(-e // CB)) // NW)
    cw = cw + (cw % 2)
    n_chunks = cw * NW
    e_pad = n_chunks * CB

    ei_pad = jnp.concatenate(
        [edge_index, jnp.zeros((2, e_pad - e), jnp.int32)], axis=1)
    rows2d = ei_pad[0].reshape(n_chunks, CB)
    cols2d = ei_pad[1].reshape(n_chunks, CB)
    x_pad = jnp.zeros((n_pad, din), jnp.float32).at[:n].set(x)

    prep = _build_sc_prep(n_pad, n_chunks, dummy)
    prop = _build_sc_prop(n_pad, n_chunks)

    rowm, colm, degp = prep(rows2d, cols2d)
    g1, dis_col = _mm_scale_call(degp.reshape(NC, n_pad, 1), x_pad, W1, n)
    part1 = prop(g1, rowm, colm)
    g2 = _mid_call(part1, g1, dis_col, b1.reshape(1, 128), W2)
    part2 = prop(g2, rowm, colm)
    return _fin_call(part2, g2, dis_col, b2.reshape(1, 128), n)


# speedup vs baseline: 1.2517x; 1.2517x over previous
import functools

import jax
import jax.numpy as jnp
from jax import lax
from jax.experimental import pallas as pl
from jax.experimental.pallas import tpu as pltpu
from jax.experimental.pallas import tpu_sc as plsc

NC, NS, L = 2, 16, 16
NW = NC * NS
CB = 128


def _sc_mesh():
    return plsc.VectorSubcoreMesh(
        core_axis_name="c", subcore_axis_name="s",
        num_cores=NC, num_subcores=NS)


def _build_sc_prep(n_pad, n_chunks, dummy):
    cw = n_chunks // NW

    @functools.partial(
        pl.kernel,
        out_type=[
            jax.ShapeDtypeStruct((n_chunks, CB), jnp.int32),
            jax.ShapeDtypeStruct((n_chunks, CB), jnp.int32),
            jax.ShapeDtypeStruct((NC, n_pad), jnp.float32),
        ],
        mesh=_sc_mesh(),
        scratch_types=[
            pltpu.VMEM((cw, CB), jnp.int32),
            pltpu.VMEM((cw, CB), jnp.int32),
            pltpu.VMEM((cw, CB), jnp.int32),
            pltpu.VMEM((cw, CB), jnp.int32),
            pltpu.VMEM((CB,), jnp.float32),
            pltpu.VMEM((2048,), jnp.float32),
            pltpu.VMEM_SHARED((n_pad,), jnp.float32),
        ],
    )
    def prep(rows_hbm, cols_hbm, rowm_hbm, colm_hbm, degp_hbm,
             rbuf, cbuf, rm, cm, ones_b, zbuf, deg_sh):
        cidx = lax.axis_index("c")
        sidx = lax.axis_index("s")
        wid = sidx * NC + cidx
        base = wid * cw

        pltpu.sync_copy(rows_hbm.at[pl.ds(base, cw)], rbuf)
        pltpu.sync_copy(cols_hbm.at[pl.ds(base, cw)], cbuf)

        for k in range(CB // L):
            ones_b[pl.ds(k * L, L)] = jnp.ones((L,), jnp.float32)

        @pl.when(sidx == 0)
        def _zero_deg():
            def zb(i, carry):
                zbuf[pl.ds(i * L, L)] = jnp.zeros((L,), jnp.float32)
                return carry
            lax.fori_loop(0, 2048 // L, zb, 0)

            def zc(t, carry):
                pltpu.sync_copy(zbuf, deg_sh.at[pl.ds(t * 2048, 2048)])
                return carry
            lax.fori_loop(0, n_pad // 2048, zc, 0)

        def remap(j, carry):
            for k in range(CB // L):
                dvec = dummy + k * L + lax.broadcasted_iota(jnp.int32, (L,), 0)
                r = rbuf[j, pl.ds(k * L, L)]
                c2 = cbuf[j, pl.ds(k * L, L)]
                m = r == c2
                rm[j, pl.ds(k * L, L)] = jnp.where(m, dvec, r)
                cm[j, pl.ds(k * L, L)] = jnp.where(m, dvec, c2)
            return carry
        lax.fori_loop(0, cw, remap, 0)

        pltpu.sync_copy(rm, rowm_hbm.at[pl.ds(base, cw)])
        pltpu.sync_copy(cm, colm_hbm.at[pl.ds(base, cw)])

        plsc.subcore_barrier()

        def scat(j, carry):
            pltpu.sync_copy(ones_b, deg_sh.at[rm.at[j]], add=True)
            return carry
        lax.fori_loop(0, cw, scat, 0)

        plsc.subcore_barrier()

        @pl.when(sidx == 0)
        def _drain():
            pltpu.sync_copy(deg_sh, degp_hbm.at[cidx])

    return prep


def _build_sc_prop(n_pad, n_chunks):
    cw = n_chunks // NW
    ch = cw // 2
    rpt = n_pad // NS

    @functools.partial(
        pl.kernel,
        out_type=jax.ShapeDtypeStruct((NC, n_pad, 128), jnp.float32),
        mesh=_sc_mesh(),
        scratch_types=[
            pltpu.VMEM((ch, CB), jnp.int32),
            pltpu.VMEM((ch, CB), jnp.int32),
            pltpu.VMEM((2, CB, 128), jnp.float32),
            pltpu.VMEM_SHARED((n_pad, 128), jnp.float32),
            pltpu.SemaphoreType.DMA,
            pltpu.SemaphoreType.DMA,
        ],
    )
    def prop(g_hbm, rowm_hbm, colm_hbm, part_hbm, rbuf, cbuf, msg,
             acc_sh, sem0, sem1):
        cidx = lax.axis_index("c")
        sidx = lax.axis_index("s")
        wid = sidx * NC + cidx
        base = wid * cw

        pltpu.sync_copy(rowm_hbm.at[pl.ds(base, ch)], rbuf)
        pltpu.sync_copy(colm_hbm.at[pl.ds(base, ch)], cbuf)
        pltpu.async_copy(g_hbm.at[rbuf.at[0]], msg.at[0], sem0)

        def zb(i, carry):
            for k in range(128 // L):
                msg[1, i, pl.ds(k * L, L)] = jnp.zeros((L,), jnp.float32)
            return carry
        lax.fori_loop(0, CB, zb, 0)

        def zc(t, carry):
            pltpu.sync_copy(msg.at[1], acc_sh.at[pl.ds(sidx * rpt + t * CB, CB)])
            return carry
        lax.fori_loop(0, rpt // CB, zc, 0)

        plsc.subcore_barrier()

        for h in range(2):
            if h:
                pltpu.sync_copy(rowm_hbm.at[pl.ds(base + h * ch, ch)], rbuf)
                pltpu.sync_copy(colm_hbm.at[pl.ds(base + h * ch, ch)], cbuf)
                pltpu.async_copy(g_hbm.at[rbuf.at[0]], msg.at[0], sem0)

            def body(jj, carry):
                j = jj * 2
                pltpu.async_copy(g_hbm.at[rbuf.at[j + 1]], msg.at[1], sem1)
                pltpu.make_async_copy(
                    g_hbm.at[rbuf.at[j]], msg.at[0], sem0).wait()
                pltpu.sync_copy(msg.at[0], acc_sh.at[cbuf.at[j]], add=True)

                @pl.when(jj < ch // 2 - 1)
                def _():
                    pltpu.async_copy(
                        g_hbm.at[rbuf.at[j + 2]], msg.at[0], sem0)

                pltpu.make_async_copy(
                    g_hbm.at[rbuf.at[j + 1]], msg.at[1], sem1).wait()
                pltpu.sync_copy(msg.at[1], acc_sh.at[cbuf.at[j + 1]], add=True)
                return carry
            lax.fori_loop(0, ch // 2, body, 0)

        plsc.subcore_barrier()

        pltpu.sync_copy(acc_sh.at[pl.ds(sidx * rpt, rpt)],
                        part_hbm.at[cidx, pl.ds(sidx * rpt, rpt)])

    return prop


def _mm_scale_call(degp3, x_pad, w, n, bm=2048):
    n_pad = x_pad.shape[0]
    d_out = w.shape[1]

    def body(d_ref, x_ref, w_ref, g_ref, dis_ref):
        i = pl.program_id(0)
        deg = d_ref[0] + d_ref[1] + 1.0
        idx = lax.broadcasted_iota(jnp.int32, (bm, 1), 0) + i * bm
        dis = jnp.where(idx < n, lax.rsqrt(deg), 0.0)
        dis_ref[...] = dis
        g_ref[...] = jnp.dot(x_ref[...], w_ref[...],
                             preferred_element_type=jnp.float32) * dis

    return pl.pallas_call(
        body,
        grid=(n_pad // bm,),
        in_specs=[
            pl.BlockSpec((NC, bm, 1), lambda i: (0, i, 0)),
            pl.BlockSpec((bm, x_pad.shape[1]), lambda i: (i, 0)),
            pl.BlockSpec(w.shape, lambda i: (0, 0)),
        ],
        out_specs=[
            pl.BlockSpec((bm, d_out), lambda i: (i, 0)),
            pl.BlockSpec((bm, 1), lambda i: (i, 0)),
        ],
        out_shape=[
            jax.ShapeDtypeStruct((n_pad, d_out), jnp.float32),
            jax.ShapeDtypeStruct((n_pad, 1), jnp.float32),
        ],
    )(degp3, x_pad, w)


def _mid_call(part, g1, dis_col, b1r, w2, bm=2048):
    n_pad = g1.shape[0]
    d_out = w2.shape[1]

    def body(p_ref, g_ref, d_ref, b_ref, w_ref, o_ref):
        agg = p_ref[0] + p_ref[1] + g_ref[...]
        h1 = jnp.maximum(agg * d_ref[...] + b_ref[...], 0.0)
        o_ref[...] = jnp.dot(h1, w_ref[...],
                             preferred_element_type=jnp.float32) * d_ref[...]

    return pl.pallas_call(
        body,
        grid=(n_pad // bm,),
        in_specs=[
            pl.BlockSpec((NC, bm, 128), lambda i: (0, i, 0)),
            pl.BlockSpec((bm, 128), lambda i: (i, 0)),
            pl.BlockSpec((bm, 1), lambda i: (i, 0)),
            pl.BlockSpec((1, 128), lambda i: (0, 0)),
            pl.BlockSpec(w2.shape, lambda i: (0, 0)),
        ],
        out_specs=pl.BlockSpec((bm, d_out), lambda i: (i, 0)),
        out_shape=jax.ShapeDtypeStruct((n_pad, d_out), jnp.float32),
    )(part, g1, dis_col, b1r, w2)


def _fin_call(part, g2, dis_col, b2r, n, bm=2048):
    n_pad = g2.shape[0]

    def body(p_ref, g_ref, d_ref, b_ref, o_ref):
        o_ref[...] = (p_ref[0] + p_ref[1] + g_ref[...]) * d_ref[...] + b_ref[...]

    return pl.pallas_call(
        body,
        grid=(pl.cdiv(n, bm),),
        in_specs=[
            pl.BlockSpec((NC, bm, 128), lambda i: (0, i, 0)),
            pl.BlockSpec((bm, 128), lambda i: (i, 0)),
            pl.BlockSpec((bm, 1), lambda i: (i, 0)),
            pl.BlockSpec((1, 128), lambda i: (0, 0)),
        ],
        out_specs=pl.BlockSpec((bm, 128), lambda i: (i, 0)),
        out_shape=jax.ShapeDtypeStruct((n, 128), jnp.float32),
    )(part, g2, dis_col, b2r)


def kernel(x, edge_index, cache_name, W1, b1, W2, b2):
    del cache_name
    n, din = x.shape
    e = edge_index.shape[1]
    dummy = n
    n_pad = ((n + CB + 1023) // 1024) * 1024

    cw = -(-(-(-e // CB)) // NW)
    cw = cw + (cw % 2)
    n_chunks = cw * NW
    e_pad = n_chunks * CB

    ei_pad = jnp.concatenate(
        [edge_index, jnp.zeros((2, e_pad - e), jnp.int32)], axis=1)
    rows2d = ei_pad[0].reshape(n_chunks, CB)
    cols2d = ei_pad[1].reshape(n_chunks, CB)
    x_pad = jnp.zeros((n_pad, din), jnp.float32).at[:n].set(x)

    prep = _build_sc_prep(n_pad, n_chunks, dummy)
    prop = _build_sc_prop(n_pad, n_chunks)

    rowm, colm, degp = prep(rows2d, cols2d)
    g1, dis_col = _mm_scale_call(degp.reshape(NC, n_pad, 1), x_pad, W1, n)
    part1 = prop(g1, rowm, colm)
    g2 = _mid_call(part1, g1, dis_col, b1.reshape(1, 128), W2)
    part2 = prop(g2, rowm, colm)
    return _fin_call(part2, g2, dis_col, b2.reshape(1, 128), n)


# speedup vs baseline: 1.2652x; 1.0108x over previous
import functools

import jax
import jax.numpy as jnp
from jax import lax
from jax.experimental import pallas as pl
from jax.experimental.pallas import tpu as pltpu
from jax.experimental.pallas import tpu_sc as plsc

NC, NS, L = 2, 16, 16
NW = NC * NS
CB = 128


def _sc_mesh():
    return plsc.VectorSubcoreMesh(
        core_axis_name="c", subcore_axis_name="s",
        num_cores=NC, num_subcores=NS)


def _build_sc_prep(n_pad, n_chunks, dummy):
    cw = n_chunks // NW

    @functools.partial(
        pl.kernel,
        out_type=[
            jax.ShapeDtypeStruct((n_chunks, CB), jnp.int32),
            jax.ShapeDtypeStruct((n_chunks, CB), jnp.int32),
            jax.ShapeDtypeStruct((NC, n_pad), jnp.float32),
        ],
        mesh=_sc_mesh(),
        scratch_types=[
            pltpu.VMEM((cw, CB), jnp.int32),
            pltpu.VMEM((cw, CB), jnp.int32),
            pltpu.VMEM((cw, CB), jnp.int32),
            pltpu.VMEM((cw, CB), jnp.int32),
            pltpu.VMEM((CB,), jnp.float32),
            pltpu.VMEM((2048,), jnp.float32),
            pltpu.VMEM_SHARED((n_pad,), jnp.float32),
        ],
    )
    def prep(rows_hbm, cols_hbm, rowm_hbm, colm_hbm, degp_hbm,
             rbuf, cbuf, rm, cm, ones_b, zbuf, deg_sh):
        cidx = lax.axis_index("c")
        sidx = lax.axis_index("s")
        wid = sidx * NC + cidx
        base = wid * cw

        pltpu.sync_copy(rows_hbm.at[pl.ds(base, cw)], rbuf)
        pltpu.sync_copy(cols_hbm.at[pl.ds(base, cw)], cbuf)

        for k in range(CB // L):
            ones_b[pl.ds(k * L, L)] = jnp.ones((L,), jnp.float32)

        @pl.when(sidx == 0)
        def _zero_deg():
            def zb(i, carry):
                zbuf[pl.ds(i * L, L)] = jnp.zeros((L,), jnp.float32)
                return carry
            lax.fori_loop(0, 2048 // L, zb, 0)

            def zc(t, carry):
                pltpu.sync_copy(zbuf, deg_sh.at[pl.ds(t * 2048, 2048)])
                return carry
            lax.fori_loop(0, n_pad // 2048, zc, 0)

        def remap(j, carry):
            for k in range(CB // L):
                dvec = dummy + k * L + lax.broadcasted_iota(jnp.int32, (L,), 0)
                r = rbuf[j, pl.ds(k * L, L)]
                c2 = cbuf[j, pl.ds(k * L, L)]
                m = r == c2
                rm[j, pl.ds(k * L, L)] = jnp.where(m, dvec, r)
                cm[j, pl.ds(k * L, L)] = jnp.where(m, dvec, c2)
            return carry
        lax.fori_loop(0, cw, remap, 0)

        pltpu.sync_copy(rm, rowm_hbm.at[pl.ds(base, cw)])
        pltpu.sync_copy(cm, colm_hbm.at[pl.ds(base, cw)])

        plsc.subcore_barrier()

        def scat(j, carry):
            pltpu.sync_copy(ones_b, deg_sh.at[rm.at[j]], add=True)
            return carry
        lax.fori_loop(0, cw, scat, 0)

        plsc.subcore_barrier()

        @pl.when(sidx == 0)
        def _drain():
            pltpu.sync_copy(deg_sh, degp_hbm.at[cidx])

    return prep


def _build_sc_prop(n_pad, n_chunks):
    cw = n_chunks // NW
    ch = cw // 2
    rpt = n_pad // NS

    @functools.partial(
        pl.kernel,
        out_type=jax.ShapeDtypeStruct((NC, n_pad, 128), jnp.float32),
        mesh=_sc_mesh(),
        scratch_types=[
            pltpu.VMEM((ch, CB), jnp.int32),
            pltpu.VMEM((ch, CB), jnp.int32),
            pltpu.VMEM((2, CB, 128), jnp.float32),
            pltpu.VMEM_SHARED((n_pad, 128), jnp.float32),
            pltpu.SemaphoreType.DMA,
            pltpu.SemaphoreType.DMA,
        ],
    )
    def prop(g_hbm, rowm_hbm, colm_hbm, part_hbm, rbuf, cbuf, msg,
             acc_sh, sem0, sem1):
        cidx = lax.axis_index("c")
        sidx = lax.axis_index("s")
        wid = sidx * NC + cidx
        base = wid * cw

        pltpu.sync_copy(rowm_hbm.at[pl.ds(base, ch)], rbuf)
        pltpu.sync_copy(colm_hbm.at[pl.ds(base, ch)], cbuf)
        pltpu.async_copy(g_hbm.at[rbuf.at[0]], msg.at[0], sem0)

        def zb(i, carry):
            for k in range(128 // L):
                msg[1, i, pl.ds(k * L, L)] = jnp.zeros((L,), jnp.float32)
            return carry
        lax.fori_loop(0, CB, zb, 0)

        def zc(t, carry):
            pltpu.sync_copy(msg.at[1], acc_sh.at[pl.ds(sidx * rpt + t * CB, CB)])
            return carry
        lax.fori_loop(0, rpt // CB, zc, 0)

        plsc.subcore_barrier()

        for h in range(2):
            if h:
                pltpu.sync_copy(rowm_hbm.at[pl.ds(base + h * ch, ch)], rbuf)
                pltpu.sync_copy(colm_hbm.at[pl.ds(base + h * ch, ch)], cbuf)
                pltpu.async_copy(g_hbm.at[rbuf.at[0]], msg.at[0], sem0)

            def body(jj, carry):
                j = jj * 2
                pltpu.async_copy(g_hbm.at[rbuf.at[j + 1]], msg.at[1], sem1)
                pltpu.make_async_copy(
                    g_hbm.at[rbuf.at[j]], msg.at[0], sem0).wait()
                pltpu.sync_copy(msg.at[0], acc_sh.at[cbuf.at[j]], add=True)

                @pl.when(jj < ch // 2 - 1)
                def _():
                    pltpu.async_copy(
                        g_hbm.at[rbuf.at[j + 2]], msg.at[0], sem0)

                pltpu.make_async_copy(
                    g_hbm.at[rbuf.at[j + 1]], msg.at[1], sem1).wait()
                pltpu.sync_copy(msg.at[1], acc_sh.at[cbuf.at[j + 1]], add=True)
                return carry
            lax.fori_loop(0, ch // 2, body, 0)

        plsc.subcore_barrier()

        pltpu.sync_copy(acc_sh.at[pl.ds(sidx * rpt, rpt)],
                        part_hbm.at[cidx, pl.ds(sidx * rpt, rpt)])

    return prop


def _mm_scale_call(degp3, x_pad, w, n, bm=5120):
    n_pad = x_pad.shape[0]
    d_out = w.shape[1]

    def body(d_ref, x_ref, w_ref, g_ref, dis_ref):
        i = pl.program_id(0)
        deg = d_ref[0] + d_ref[1] + 1.0
        idx = lax.broadcasted_iota(jnp.int32, (bm, 1), 0) + i * bm
        dis = jnp.where(idx < n, lax.rsqrt(deg), 0.0)
        dis_ref[...] = dis
        g_ref[...] = jnp.dot(x_ref[...], w_ref[...],
                             preferred_element_type=jnp.float32) * dis

    return pl.pallas_call(
        body,
        grid=(n_pad // bm,),
        in_specs=[
            pl.BlockSpec((NC, bm, 1), lambda i: (0, i, 0)),
            pl.BlockSpec((bm, x_pad.shape[1]), lambda i: (i, 0)),
            pl.BlockSpec(w.shape, lambda i: (0, 0)),
        ],
        out_specs=[
            pl.BlockSpec((bm, d_out), lambda i: (i, 0)),
            pl.BlockSpec((bm, 1), lambda i: (i, 0)),
        ],
        out_shape=[
            jax.ShapeDtypeStruct((n_pad, d_out), jnp.float32),
            jax.ShapeDtypeStruct((n_pad, 1), jnp.float32),
        ],
    )(degp3, x_pad, w)


def _mid_call(part, g1, dis_col, b1r, w2, bm=5120):
    n_pad = g1.shape[0]
    d_out = w2.shape[1]

    def body(p_ref, g_ref, d_ref, b_ref, w_ref, o_ref):
        agg = p_ref[0] + p_ref[1] + g_ref[...]
        h1 = jnp.maximum(agg * d_ref[...] + b_ref[...], 0.0)
        o_ref[...] = jnp.dot(h1, w_ref[...],
                             preferred_element_type=jnp.float32) * d_ref[...]

    return pl.pallas_call(
        body,
        grid=(n_pad // bm,),
        in_specs=[
            pl.BlockSpec((NC, bm, 128), lambda i: (0, i, 0)),
            pl.BlockSpec((bm, 128), lambda i: (i, 0)),
            pl.BlockSpec((bm, 1), lambda i: (i, 0)),
            pl.BlockSpec((1, 128), lambda i: (0, 0)),
            pl.BlockSpec(w2.shape, lambda i: (0, 0)),
        ],
        out_specs=pl.BlockSpec((bm, d_out), lambda i: (i, 0)),
        out_shape=jax.ShapeDtypeStruct((n_pad, d_out), jnp.float32),
    )(part, g1, dis_col, b1r, w2)


def _fin_call(part, g2, dis_col, b2r, n, bm=5120):
    n_pad = g2.shape[0]

    def body(p_ref, g_ref, d_ref, b_ref, o_ref):
        o_ref[...] = (p_ref[0] + p_ref[1] + g_ref[...]) * d_ref[...] + b_ref[...]

    return pl.pallas_call(
        body,
        grid=(pl.cdiv(n, bm),),
        in_specs=[
            pl.BlockSpec((NC, bm, 128), lambda i: (0, i, 0)),
            pl.BlockSpec((bm, 128), lambda i: (i, 0)),
            pl.BlockSpec((bm, 1), lambda i: (i, 0)),
            pl.BlockSpec((1, 128), lambda i: (0, 0)),
        ],
        out_specs=pl.BlockSpec((bm, 128), lambda i: (i, 0)),
        out_shape=jax.ShapeDtypeStruct((n, 128), jnp.float32),
    )(part, g2, dis_col, b2r)


def kernel(x, edge_index, cache_name, W1, b1, W2, b2):
    del cache_name
    n, din = x.shape
    e = edge_index.shape[1]
    dummy = n
    n_pad = ((n + CB + 1023) // 1024) * 1024

    cw = -(-(-(-e // CB)) // NW)
    cw = cw + (cw % 2)
    n_chunks = cw * NW
    e_pad = n_chunks * CB

    ei_pad = jnp.concatenate(
        [edge_index, jnp.zeros((2, e_pad - e), jnp.int32)], axis=1)
    rows2d = ei_pad[0].reshape(n_chunks, CB)
    cols2d = ei_pad[1].reshape(n_chunks, CB)
    x_pad = jnp.zeros((n_pad, din), jnp.float32).at[:n].set(x)

    prep = _build_sc_prep(n_pad, n_chunks, dummy)
    prop = _build_sc_prop(n_pad, n_chunks)

    rowm, colm, degp = prep(rows2d, cols2d)
    g1, dis_col = _mm_scale_call(degp.reshape(NC, n_pad, 1), x_pad, W1, n)
    part1 = prop(g1, rowm, colm)
    g2 = _mid_call(part1, g1, dis_col, b1.reshape(1, 128), W2)
    part2 = prop(g2, rowm, colm)
    return _fin_call(part2, g2, dis_col, b2.reshape(1, 128), n)
